# baseline (speedup 1.0000x reference)
import functools

import jax
import jax.numpy as jnp
from jax import lax
from jax.experimental import pallas as pl
from jax.experimental.pallas import tpu as pltpu
from jax.experimental.pallas import tpu_sc as plsc

B = 16
LX = 1024
LC = 1024
LT = 2048
D = 64
PAD = -999.0

L = 16
NC = 2
NS = 16
NW = NC * NS
ROWS = (B * LC) // NW
BLK = ROWS * D
WIN = ROWS + 528


def _sc_body(x_ref, t_ref, lenx_ref, lenc_ref, out_ref,
             block, win, lenxv, lencv, t0b, xb0, xbl, sem):
    wid = lax.axis_index("s") * NC + lax.axis_index("c")
    i = wid // 2
    jbase = (wid % 2) * (LC // 2)

    pltpu.async_copy(lenx_ref, lenxv.at[pl.ds(0, L)], sem).wait()
    pltpu.async_copy(lenc_ref, lencv.at[pl.ds(0, L)], sem).wait()
    lx = lenxv[pl.ds(i, L)][0]
    lc = lencv[pl.ds(i, L)][0]

    a = jnp.minimum((lx - 8) & -8, LT - WIN)
    wstart = pl.multiple_of(i * LT + a, 8)
    pltpu.async_copy(t_ref.at[pl.ds(wstart, WIN)], win, sem).wait()
    pltpu.async_copy(t_ref.at[pl.ds(pl.multiple_of(i * LT, 8), L)], t0b,
                     sem).wait()
    pltpu.async_copy(x_ref.at[pl.ds(pl.multiple_of(i * (LX * D), 8), L)], xb0,
                     sem).wait()
    pltpu.async_copy(
        x_ref.at[pl.ds(pl.multiple_of(i * (LX * D) + (lx - 1) * D, 8), L)],
        xbl, sem).wait()

    r = lx - a
    t0 = jnp.broadcast_to(t0b[...][0], (L,))
    beta = jnp.broadcast_to(xb0[...][0], (L,))
    x_last = jnp.broadcast_to(xbl[...][0], (L,))
    t_last = jnp.broadcast_to(win[pl.ds(r - 1, L)][0], (L,))
    slope = (x_last - beta) / (t_last - t0)

    fill = jnp.full((L,), PAD, jnp.float32)

    def _fill_body(m, carry):
        base = m * (16 * L)
        for t in range(16):
            block[pl.ds(base + t * L, L)] = fill
        return carry

    lax.fori_loop(0, BLK // (16 * L), _fill_body, 0)

    lane = lax.broadcasted_iota(jnp.int32, (L,), 0)
    for k in range(ROWS // L):
        fut = win[pl.ds(r + jbase + k * L, L)]
        pred = slope * (fut - t0) + beta
        jv = jbase + k * L + lane
        val = jnp.where(jv < lc, pred, PAD)
        plsc.store_scatter(block, [(k * L + lane) * D], val)

    pltpu.async_copy(block, out_ref.at[pl.ds(wid * BLK, BLK)], sem).wait()


@functools.partial(jax.jit, static_argnames=("interpret",))
def _run(x, time, len_x, len_context, interpret=False):
    fn = pl.kernel(
        _sc_body,
        out_type=jax.ShapeDtypeStruct((B * LC * D,), jnp.float32),
        mesh=plsc.VectorSubcoreMesh(core_axis_name="c", subcore_axis_name="s"),
        compiler_params=pltpu.CompilerParams(needs_layout_passes=False),
        scratch_types=[
            pltpu.VMEM((BLK,), jnp.float32),
            pltpu.VMEM((WIN,), jnp.float32),
            pltpu.VMEM((2 * L,), jnp.int32),
            pltpu.VMEM((2 * L,), jnp.int32),
            pltpu.VMEM((L,), jnp.float32),
            pltpu.VMEM((L,), jnp.float32),
            pltpu.VMEM((L,), jnp.float32),
            pltpu.SemaphoreType.DMA,
        ],
        interpret=interpret,
    )
    out = fn(x.reshape(B * LX * D), time.reshape(B * LT),
             len_x.astype(jnp.int32), len_context.astype(jnp.int32))
    return out.reshape(B, LC, D)


def kernel(x, time, context, len_x, len_context):
    return _run(x, time, len_x, len_context)


# speedup vs baseline: 2.2282x; 2.2282x over previous
import functools

import jax
import jax.numpy as jnp
from jax.experimental import pallas as pl
from jax.experimental.pallas import tpu as pltpu

B = 16
LX = 1024
LC = 1024
LT = 2048
D = 64
PAD = -999.0
RPS = 2


def _one_row(lx, lc, x0_blk, xl_blk, trow):
    beta = x0_blk[0, 0]

    r = (lx - 1) % 8
    row_ids = jax.lax.broadcasted_iota(jnp.int32, (8, D), 0)
    col_ids = jax.lax.broadcasted_iota(jnp.int32, (8, D), 1)
    x_last = jnp.sum(jnp.where((row_ids == r) & (col_ids == 0), xl_blk, 0.0))

    t0 = trow[0, 0]
    rot = pltpu.roll(trow, LT - lx, 1)
    t_last = rot[0, LT - 1] - t0
    slope = (x_last - beta) / t_last

    fut = rot[:, :LC] - t0
    pred = slope * fut + beta

    pos = jax.lax.broadcasted_iota(jnp.int32, (LC, 1), 0)
    col = jnp.where(pos < lc, pred.reshape(LC, 1), PAD)

    d_ids = jax.lax.broadcasted_iota(jnp.int32, (LC, D), 1)
    return jnp.where(d_ids == 0, col, PAD)


def _row_kernel(lx_ref, lc_ref, *refs):
    x_refs = refs[: 2 * RPS]
    t_ref = refs[2 * RPS]
    o_ref = refs[2 * RPS + 1]
    g = pl.program_id(0)
    for k in range(RPS):
        i = RPS * g + k
        o_ref[k] = _one_row(lx_ref[i], lc_ref[i], x_refs[2 * k][0],
                            x_refs[2 * k + 1][0], t_ref[pl.ds(i, 1)])


def _x_specs():
    specs = []
    for k in range(RPS):
        specs.append(
            pl.BlockSpec((1, 8, D),
                         lambda g, lx, lc, k=k: (RPS * g + k, 0, 0)))
        specs.append(
            pl.BlockSpec(
                (1, 8, D),
                lambda g, lx, lc, k=k:
                (RPS * g + k, (lx[RPS * g + k] - 1) // 8, 0)))
    return specs


@functools.partial(jax.jit, static_argnames=("interpret",))
def _run(x, time, len_x, len_context, interpret=False):
    grid_spec = pltpu.PrefetchScalarGridSpec(
        num_scalar_prefetch=2,
        grid=(B // RPS,),
        in_specs=_x_specs() + [pl.BlockSpec((B, LT), lambda g, lx, lc: (0, 0))],
        out_specs=pl.BlockSpec((RPS, LC, D), lambda g, lx, lc: (g, 0, 0)),
    )
    return pl.pallas_call(
        _row_kernel,
        grid_spec=grid_spec,
        out_shape=jax.ShapeDtypeStruct((B, LC, D), jnp.float32),
        interpret=interpret,
    )(len_x, len_context, *([x] * (2 * RPS)), time)


def kernel(x, time, context, len_x, len_context):
    return _run(x, time, len_x, len_context)
